# baseline (speedup 1.0000x reference)
import jax
import jax.numpy as jnp
from jax.experimental import pallas as pl
from jax.experimental.pallas import tpu as pltpu

_L = 2048
_DM = 768
_DI = 1536
_DS = 16
_DC = 4
_DTR = 48

_CH = 128
_HALO = 8


def _silu(x):
    return x * jax.nn.sigmoid(x)


def _softplus(x):
    return jnp.maximum(x, 0.0) + jnp.log1p(jnp.exp(-jnp.abs(x)))


def _body(x_ref, xh_ref, win_ref, cwt_ref, cb_ref, wx_ref, wdt_ref, bdt_ref,
          at_ref, d_ref, wout_ref, out_ref,
          dA_ref, dBu_ref, hc_ref, dt0_ref, q_ref, m2_ref, m2t_ref, p_ref):
    i = pl.program_id(0)
    ni = pl.num_programs(0)
    f32 = jnp.float32

    @pl.when(i == 0)
    def _init():
        hc_ref[...] = jnp.zeros_like(hc_ref)
        jn = jax.lax.broadcasted_iota(jnp.int32, (_DS, _CH * _DS), 1)
        nn = jax.lax.broadcasted_iota(jnp.int32, (_DS, _CH * _DS), 0)
        q_ref[...] = ((jn & (_DS - 1)) == nn).astype(f32)
        jt = jax.lax.broadcasted_iota(jnp.int32, (_CH, _CH * _DS), 1)
        tt = jax.lax.broadcasted_iota(jnp.int32, (_CH, _CH * _DS), 0)
        m2_ref[...] = ((jt // _DS) == tt).astype(f32)
        jt2 = jax.lax.broadcasted_iota(jnp.int32, (_CH * _DS, _CH), 0)
        tt2 = jax.lax.broadcasted_iota(jnp.int32, (_CH * _DS, _CH), 1)
        m2t_ref[...] = ((jt2 // _DS) == tt2).astype(f32)
        jp = jax.lax.broadcasted_iota(jnp.int32, (_CH * _DS, _DS), 0)
        np_ = jax.lax.broadcasted_iota(jnp.int32, (_CH * _DS, _DS), 1)
        p_ref[...] = ((jp & (_DS - 1)) == np_).astype(f32)

    xa = jnp.concatenate([xh_ref[...], x_ref[...]], axis=0)
    proj = jnp.dot(xa, win_ref[...], preferred_element_type=f32)
    hp = proj[:, :_DI]
    gate = proj[_HALO:, _DI:]
    row = jax.lax.broadcasted_iota(jnp.int32, (_HALO + _CH, 1), 0)
    keep = jnp.logical_or(row >= _HALO, i != ni - 1)
    hp = jnp.where(keep, hp, 0.0)
    conv = cb_ref[...] + (hp[_HALO - 3:_HALO - 3 + _CH] * cwt_ref[0:1]
                          + hp[_HALO - 2:_HALO - 2 + _CH] * cwt_ref[1:2]
                          + hp[_HALO - 1:_HALO - 1 + _CH] * cwt_ref[2:3]
                          + hp[_HALO:_HALO + _CH] * cwt_ref[3:4])
    u_c = _silu(conv)
    sg = _silu(gate)
    ssm = jnp.dot(u_c, wx_ref[...], preferred_element_type=f32)
    bc = ssm[:, _DTR:_DTR + 2 * _DS]
    dt_c = _softplus(
        jnp.dot(ssm[:, :_DTR], wdt_ref[...], preferred_element_type=f32)
        + bdt_ref[...])

    nrow = jnp.where(i == 0, 0.0, dt0_ref[0:1])
    dts_c = jnp.concatenate([dt_c[1:], nrow], axis=0)
    dt0_ref[...] = dt_c[0:_HALO]
    argx = jnp.dot(m2t_ref[...], dts_c, preferred_element_type=f32)
    dA_ref[...] = jnp.exp(
        argx.reshape(_CH, _DS, _DI) * at_ref[...][None, :, :])
    du = dt_c * u_c
    mbt = jnp.dot(p_ref[...], bc[:, :_DS].T,
                  preferred_element_type=f32) * m2t_ref[...]
    dBu_ref[...] = jnp.dot(mbt, du,
                           preferred_element_type=f32).reshape(_CH, _DS, _DI)

    def body(k, h):
        t = _CH - 1 - k
        h2 = dBu_ref[t] + dA_ref[t] * h
        dBu_ref[t] = h2
        return h2

    h = jax.lax.fori_loop(0, _CH, body, hc_ref[...], unroll=16)
    hc_ref[...] = h

    mc = jnp.dot(bc[:, _DS:], q_ref[...], preferred_element_type=f32) * m2_ref[...]
    h2d = dBu_ref[...].reshape(_CH * _DS, _DI)
    y = 2.0 * jnp.dot(mc, h2d, preferred_element_type=f32)
    y = (y + u_c * d_ref[...]) * sg
    out_ref[...] = jnp.dot(y, wout_ref[...], preferred_element_type=f32)


def kernel(input_states, context_len, W_in, conv_w, conv_b, W_x, W_dt, b_dt,
           A_log, D, W_out):
    f32 = jnp.float32
    x = input_states[0]
    cwt = conv_w.T
    cb = conv_b[None, :]
    bdt = b_dt[None, :]
    at = (-jnp.exp(A_log)).T
    drow = D[None, :]

    grid = _L // _CH

    def rev(i):
        return (grid - 1 - i, 0)

    def rev_halo(i):
        return (jnp.maximum((grid - 1 - i) * (_CH // _HALO) - 1, 0), 0)

    const = lambda i: (0, 0)
    out = pl.pallas_call(
        _body,
        grid=(grid,),
        in_specs=[
            pl.BlockSpec((_CH, _DM), rev),
            pl.BlockSpec((_HALO, _DM), rev_halo),
            pl.BlockSpec((_DM, 2 * _DI), const),
            pl.BlockSpec((_DC, _DI), const),
            pl.BlockSpec((1, _DI), const),
            pl.BlockSpec((_DI, _DTR + 2 * _DS), const),
            pl.BlockSpec((_DTR, _DI), const),
            pl.BlockSpec((1, _DI), const),
            pl.BlockSpec((_DS, _DI), const),
            pl.BlockSpec((1, _DI), const),
            pl.BlockSpec((_DI, _DM), const),
        ],
        out_specs=pl.BlockSpec((_CH, _DM), rev),
        out_shape=jax.ShapeDtypeStruct((_L, _DM), f32),
        scratch_shapes=[
            pltpu.VMEM((_CH, _DS, _DI), f32),
            pltpu.VMEM((_CH, _DS, _DI), f32),
            pltpu.VMEM((_DS, _DI), f32),
            pltpu.VMEM((_HALO, _DI), f32),
            pltpu.VMEM((_DS, _CH * _DS), f32),
            pltpu.VMEM((_CH, _CH * _DS), f32),
            pltpu.VMEM((_CH * _DS, _CH), f32),
            pltpu.VMEM((_CH * _DS, _DS), f32),
        ],
    )(x, x, W_in.astype(jnp.bfloat16), cwt, cb, W_x.astype(jnp.bfloat16),
      W_dt.astype(jnp.bfloat16), bdt, at, drow, W_out.astype(jnp.bfloat16))

    valid = jnp.where(jnp.asarray(context_len) <= 2, f32(1.0), f32(jnp.nan))
    return out[None] * valid


# speedup vs baseline: 1.0569x; 1.0569x over previous
import jax
import jax.numpy as jnp
from jax.experimental import pallas as pl
from jax.experimental.pallas import tpu as pltpu

_L = 2048
_DM = 768
_DI = 1536
_DS = 16
_DC = 4
_DTR = 48

_CH = 128
_HALO = 8


def _silu(x):
    return x * jax.nn.sigmoid(x)


def _softplus(x):
    return jnp.maximum(x, 0.0) + jnp.log1p(jnp.exp(-jnp.abs(x)))


def _body(x_ref, xh_ref, win_ref, cwt_ref, cb_ref, wx_ref, wdt_ref, bdt_ref,
          at_ref, d_ref, wout_ref, out_ref,
          dA_ref, dBu_ref, hc_ref, dt0_ref, q_ref, m2_ref, m2t_ref):
    i = pl.program_id(0)
    ni = pl.num_programs(0)
    f32 = jnp.float32

    @pl.when(i == 0)
    def _init():
        hc_ref[...] = jnp.zeros_like(hc_ref)
        jn = jax.lax.broadcasted_iota(jnp.int32, (_DS, _CH * _DS), 1)
        nn = jax.lax.broadcasted_iota(jnp.int32, (_DS, _CH * _DS), 0)
        q_ref[...] = ((jn & (_DS - 1)) == nn).astype(jnp.bfloat16)
        jt = jax.lax.broadcasted_iota(jnp.int32, (_CH, _CH * _DS), 1)
        tt = jax.lax.broadcasted_iota(jnp.int32, (_CH, _CH * _DS), 0)
        m2_ref[...] = ((jt // _DS) == tt).astype(f32)
        jt2 = jax.lax.broadcasted_iota(jnp.int32, (_CH * _DS, _CH), 0)
        tt2 = jax.lax.broadcasted_iota(jnp.int32, (_CH * _DS, _CH), 1)
        m2t_ref[...] = ((jt2 // _DS) == tt2).astype(jnp.bfloat16)

    xa = jnp.concatenate([xh_ref[...], x_ref[...]], axis=0)
    proj = jnp.dot(xa, win_ref[...], preferred_element_type=f32)
    hp = proj[:, :_DI]
    gate = proj[_HALO:, _DI:]
    row = jax.lax.broadcasted_iota(jnp.int32, (_HALO + _CH, 1), 0)
    keep = jnp.logical_or(row >= _HALO, i != ni - 1)
    hp = jnp.where(keep, hp, 0.0)
    conv = cb_ref[...] + (hp[_HALO - 3:_HALO - 3 + _CH] * cwt_ref[0:1]
                          + hp[_HALO - 2:_HALO - 2 + _CH] * cwt_ref[1:2]
                          + hp[_HALO - 1:_HALO - 1 + _CH] * cwt_ref[2:3]
                          + hp[_HALO:_HALO + _CH] * cwt_ref[3:4])
    u_c = _silu(conv)
    sg = _silu(gate)
    ssm = jnp.dot(u_c, wx_ref[...], preferred_element_type=f32)
    bc = ssm[:, _DTR:_DTR + 2 * _DS]
    dt_c = _softplus(
        jnp.dot(ssm[:, :_DTR], wdt_ref[...], preferred_element_type=f32)
        + bdt_ref[...])

    nrow = jnp.where(i == 0, 0.0, dt0_ref[0:1])
    dts_c = jnp.concatenate([dt_c[1:], nrow], axis=0)
    dt0_ref[...] = dt_c[0:_HALO]
    argx = jnp.dot(m2t_ref[...], dts_c, preferred_element_type=f32)
    dA_ref[...] = jnp.exp(
        argx.reshape(_CH, _DS, _DI) * at_ref[...][None, :, :])
    du = dt_c * u_c
    dBu_ref[...] = du[:, None, :] * bc[:, :_DS][:, :, None]

    def body(k, h):
        t = _CH - 1 - k
        h2 = dBu_ref[t] + dA_ref[t] * h
        dBu_ref[t] = h2
        return h2

    h = jax.lax.fori_loop(0, _CH, body, hc_ref[...], unroll=16)
    hc_ref[...] = h

    mc = jnp.dot(bc[:, _DS:], q_ref[...], preferred_element_type=f32) * m2_ref[...]
    h2d = dBu_ref[...].reshape(_CH * _DS, _DI)
    y = 2.0 * jnp.dot(mc, h2d, preferred_element_type=f32)
    y = (y + u_c * d_ref[...]) * sg
    out_ref[...] = jnp.dot(y, wout_ref[...], preferred_element_type=f32)


def kernel(input_states, context_len, W_in, conv_w, conv_b, W_x, W_dt, b_dt,
           A_log, D, W_out):
    f32 = jnp.float32
    x = input_states[0]
    cwt = conv_w.T
    cb = conv_b[None, :]
    bdt = b_dt[None, :]
    at = (-jnp.exp(A_log)).T
    drow = D[None, :]

    grid = _L // _CH

    def rev(i):
        return (grid - 1 - i, 0)

    def rev_halo(i):
        return (jnp.maximum((grid - 1 - i) * (_CH // _HALO) - 1, 0), 0)

    const = lambda i: (0, 0)
    out = pl.pallas_call(
        _body,
        grid=(grid,),
        in_specs=[
            pl.BlockSpec((_CH, _DM), rev),
            pl.BlockSpec((_HALO, _DM), rev_halo),
            pl.BlockSpec((_DM, 2 * _DI), const),
            pl.BlockSpec((_DC, _DI), const),
            pl.BlockSpec((1, _DI), const),
            pl.BlockSpec((_DI, _DTR + 2 * _DS), const),
            pl.BlockSpec((_DTR, _DI), const),
            pl.BlockSpec((1, _DI), const),
            pl.BlockSpec((_DS, _DI), const),
            pl.BlockSpec((1, _DI), const),
            pl.BlockSpec((_DI, _DM), const),
        ],
        out_specs=pl.BlockSpec((_CH, _DM), rev),
        out_shape=jax.ShapeDtypeStruct((_L, _DM), f32),
        scratch_shapes=[
            pltpu.VMEM((_CH, _DS, _DI), f32),
            pltpu.VMEM((_CH, _DS, _DI), f32),
            pltpu.VMEM((_DS, _DI), f32),
            pltpu.VMEM((_HALO, _DI), f32),
            pltpu.VMEM((_DS, _CH * _DS), jnp.bfloat16),
            pltpu.VMEM((_CH, _CH * _DS), f32),
            pltpu.VMEM((_CH * _DS, _CH), jnp.bfloat16),
        ],
    )(x, x, W_in.astype(jnp.bfloat16), cwt, cb, W_x.astype(jnp.bfloat16),
      W_dt.astype(jnp.bfloat16), bdt, at, drow, W_out.astype(jnp.bfloat16))

    valid = jnp.where(jnp.asarray(context_len) <= 2, f32(1.0), f32(jnp.nan))
    return out[None] * valid
